# baseline (speedup 1.0000x reference)
import functools

import jax
import jax.numpy as jnp
from jax import lax
from jax.experimental import pallas as pl
from jax.experimental.pallas import tpu as pltpu
from jax.experimental.pallas import tpu_sc as plsc

NC = 2
NS = 16
NW = NC * NS
L = 16
ROWS = 128
COLS = 100000
SUB = 16
NB = ROWS // SUB
NQ = NW // NB
CW = 13 * 128
NCHUNK = 15
TAIL0 = 60 * CW
TAILW = COLS - TAIL0

_mesh = plsc.VectorSubcoreMesh(core_axis_name="c", subcore_axis_name="s")


def _scan(buf, col0, ncols, accv, acci, iota):
    def body(i, carry):
        accv, acci, cur = carry
        nv, ni = [], []
        for s in range(SUB):
            v = buf[s, pl.ds(i * L, L)]
            pred = v > accv[s]
            nv.append(jnp.where(pred, v, accv[s]))
            ni.append(jnp.where(pred, cur, acci[s]))
        return tuple(nv), tuple(ni), cur + L

    accv, acci, _ = lax.fori_loop(
        0, ncols // L, body, (accv, acci, iota + col0), unroll=1
    )
    return accv, acci


@functools.partial(
    pl.kernel,
    out_type=(
        jax.ShapeDtypeStruct((NW, L), jnp.float32),
        jax.ShapeDtypeStruct((NW, L), jnp.int32),
    ),
    mesh=_mesh,
    compiler_params=pltpu.CompilerParams(use_tc_tiling_on_sc=True),
    scratch_types=[
        pltpu.VMEM((SUB, CW), jnp.float32),
        pltpu.VMEM((SUB, CW), jnp.float32),
        pltpu.VMEM((SUB, TAILW), jnp.float32),
        pltpu.VMEM((L,), jnp.float32),
        pltpu.VMEM((L,), jnp.int32),
        pltpu.SemaphoreType.DMA,
        pltpu.SemaphoreType.DMA,
        pltpu.SemaphoreType.DMA,
    ],
)
def _argmax_sc(x_hbm, outv_hbm, outi_hbm, buf0, buf1, tailbuf, resv, resi,
               sem0, sem1, semt):
    wid = lax.axis_index("s") * NC + lax.axis_index("c")
    b = wid // NQ
    q = wid % NQ
    iota = lax.iota(jnp.int32, L)
    rows = pl.ds(b * SUB, SUB)

    def chunk_col0(i):
        return (NQ * i + q) * CW

    def chunk_src(i):
        return x_hbm.at[rows, pl.ds(chunk_col0(i), CW)]

    bufs = (buf0, buf1)
    sems = (sem0, sem1)

    pltpu.async_copy(chunk_src(0), buf0, sem0)
    pltpu.async_copy(x_hbm.at[rows, pl.ds(TAIL0, TAILW)], tailbuf, semt)

    accv = tuple(jnp.full((L,), -jnp.inf, jnp.float32) for _ in range(SUB))
    acci = tuple(jnp.zeros((L,), jnp.int32) for _ in range(SUB))

    for i in range(NCHUNK):
        s = i & 1
        if i + 1 < NCHUNK:
            pltpu.async_copy(chunk_src(i + 1), bufs[1 - s], sems[1 - s])
        pltpu.make_async_copy(chunk_src(i), bufs[s], sems[s]).wait()
        accv, acci = _scan(bufs[s], chunk_col0(i), CW, accv, acci, iota)

    pltpu.make_async_copy(
        x_hbm.at[rows, pl.ds(TAIL0, TAILW)], tailbuf, semt
    ).wait()
    accv, acci = _scan(tailbuf, TAIL0, TAILW, accv, acci, iota)

    resv_vec = jnp.zeros((L,), jnp.float32)
    resi_vec = jnp.zeros((L,), jnp.int32)
    for s in range(SUB):
        bm, bi = accv[s], acci[s]
        best_v = bm[0]
        best_i = bi[0]
        for k in range(1, L):
            pv = bm[k]
            pi = bi[k]
            pred = (pv > best_v) | ((pv == best_v) & (pi < best_i))
            best_v = jnp.where(pred, pv, best_v)
            best_i = jnp.where(pred, pi, best_i)
        resv_vec = jnp.where(iota == s, best_v, resv_vec)
        resi_vec = jnp.where(iota == s, best_i, resi_vec)

    resv[...] = resv_vec
    resi[...] = resi_vec
    pltpu.sync_copy(resv, outv_hbm.at[wid])
    pltpu.sync_copy(resi, outi_hbm.at[wid])


def kernel(m_logits):
    outv, outi = _argmax_sc(m_logits)
    v = outv.reshape(NB, NQ, L)
    i = outi.reshape(NB, NQ, L)
    bv, bi = v[:, 0], i[:, 0]
    for qq in range(1, NQ):
        pred = (v[:, qq] > bv) | ((v[:, qq] == bv) & (i[:, qq] < bi))
        bv = jnp.where(pred, v[:, qq], bv)
        bi = jnp.where(pred, i[:, qq], bi)
    return bi.reshape(ROWS, 1)


# speedup vs baseline: 1.0469x; 1.0469x over previous
import functools

import jax
import jax.numpy as jnp
from jax import lax
from jax.experimental import pallas as pl
from jax.experimental.pallas import tpu as pltpu
from jax.experimental.pallas import tpu_sc as plsc

NC = 2
NS = 16
NW = NC * NS
L = 16
ROWS = 128
COLS = 100000
SUB = 16
NB = ROWS // SUB
NQ = NW // NB
CW = 13 * 128
NCHUNK = 15
TAIL0 = 60 * CW
TAILW = COLS - TAIL0

_mesh = plsc.VectorSubcoreMesh(core_axis_name="c", subcore_axis_name="s")


def _scan(buf, col0, ncols, accv, acci, iota):
    def body(i, carry):
        accv, acci, cur = carry
        nv, ni = [], []
        for s in range(SUB):
            v = buf[s, pl.ds(i * L, L)]
            pred = v > accv[s]
            nv.append(jnp.where(pred, v, accv[s]))
            ni.append(jnp.where(pred, cur, acci[s]))
        return tuple(nv), tuple(ni), cur + L

    accv, acci, _ = lax.fori_loop(
        0, ncols // L, body, (accv, acci, iota + col0), unroll=1
    )
    return accv, acci


@functools.partial(
    pl.kernel,
    out_type=(
        jax.ShapeDtypeStruct((NW, L), jnp.float32),
        jax.ShapeDtypeStruct((NW, L), jnp.int32),
    ),
    mesh=_mesh,
    compiler_params=pltpu.CompilerParams(use_tc_tiling_on_sc=True),
    scratch_types=[
        pltpu.VMEM((SUB, CW), jnp.float32),
        pltpu.VMEM((SUB, CW), jnp.float32),
        pltpu.VMEM((SUB, CW), jnp.float32),
        pltpu.VMEM((SUB, CW), jnp.float32),
        pltpu.VMEM((SUB, TAILW), jnp.float32),
        pltpu.VMEM((L,), jnp.float32),
        pltpu.VMEM((L,), jnp.int32),
        pltpu.SemaphoreType.DMA,
        pltpu.SemaphoreType.DMA,
        pltpu.SemaphoreType.DMA,
        pltpu.SemaphoreType.DMA,
        pltpu.SemaphoreType.DMA,
    ],
)
def _argmax_sc(x_hbm, outv_hbm, outi_hbm, buf0, buf1, buf2, buf3, tailbuf,
               resv, resi, sem0, sem1, sem2, sem3, semt):
    wid = lax.axis_index("s") * NC + lax.axis_index("c")
    b = wid // NQ
    q = wid % NQ
    iota = lax.iota(jnp.int32, L)
    rows = pl.ds(b * SUB, SUB)

    def chunk_col0(i):
        return (NQ * i + q) * CW

    def chunk_src(i):
        return x_hbm.at[rows, pl.ds(chunk_col0(i), CW)]

    bufs = (buf0, buf1, buf2, buf3)
    sems = (sem0, sem1, sem2, sem3)
    DEPTH = 4

    for i in range(DEPTH - 1):
        pltpu.async_copy(chunk_src(i), bufs[i], sems[i])
    pltpu.async_copy(x_hbm.at[rows, pl.ds(TAIL0, TAILW)], tailbuf, semt)

    accv = tuple(jnp.full((L,), -jnp.inf, jnp.float32) for _ in range(SUB))
    acci = tuple(jnp.zeros((L,), jnp.int32) for _ in range(SUB))

    for i in range(NCHUNK):
        s = i % DEPTH
        if i + DEPTH - 1 < NCHUNK:
            pltpu.async_copy(
                chunk_src(i + DEPTH - 1),
                bufs[(i + DEPTH - 1) % DEPTH],
                sems[(i + DEPTH - 1) % DEPTH],
            )
        pltpu.make_async_copy(chunk_src(i), bufs[s], sems[s]).wait()
        accv, acci = _scan(bufs[s], chunk_col0(i), CW, accv, acci, iota)

    pltpu.make_async_copy(
        x_hbm.at[rows, pl.ds(TAIL0, TAILW)], tailbuf, semt
    ).wait()
    accv, acci = _scan(tailbuf, TAIL0, TAILW, accv, acci, iota)

    resv_vec = jnp.zeros((L,), jnp.float32)
    resi_vec = jnp.zeros((L,), jnp.int32)
    for s in range(SUB):
        bm, bi = accv[s], acci[s]
        best_v = bm[0]
        best_i = bi[0]
        for k in range(1, L):
            pv = bm[k]
            pi = bi[k]
            pred = (pv > best_v) | ((pv == best_v) & (pi < best_i))
            best_v = jnp.where(pred, pv, best_v)
            best_i = jnp.where(pred, pi, best_i)
        resv_vec = jnp.where(iota == s, best_v, resv_vec)
        resi_vec = jnp.where(iota == s, best_i, resi_vec)

    resv[...] = resv_vec
    resi[...] = resi_vec
    pltpu.sync_copy(resv, outv_hbm.at[wid])
    pltpu.sync_copy(resi, outi_hbm.at[wid])


def kernel(m_logits):
    outv, outi = _argmax_sc(m_logits)
    v = outv.reshape(NB, NQ, L)
    i = outi.reshape(NB, NQ, L)
    bv, bi = v[:, 0], i[:, 0]
    for qq in range(1, NQ):
        pred = (v[:, qq] > bv) | ((v[:, qq] == bv) & (i[:, qq] < bi))
        bv = jnp.where(pred, v[:, qq], bv)
        bi = jnp.where(pred, i[:, qq], bi)
    return bi.reshape(ROWS, 1)
